# baseline (speedup 1.0000x reference)
import jax
import jax.numpy as jnp
from jax.experimental import pallas as pl
from jax.experimental.pallas import tpu as pltpu

_BATCH = 16384
_DIM = 128
_TABLE_ROWS = 6
_EPS = 1e-6
_TILES = (512, 512, 1024, 2048, 2048, 2048, 2048, 2048, 2048, 1024, 512, 512)
_OFFS = tuple(sum(_TILES[:i]) for i in range(len(_TILES)))


def _ln_block(x, gamma, bias):
    mean = jnp.mean(x, axis=1, keepdims=True)
    xc = x - mean
    var = jnp.mean(xc * xc, axis=1, keepdims=True)
    inv = jax.lax.rsqrt(var + _EPS)
    return xc * inv * gamma + bias


def _mb_kernel(dom_ref, x_hbm, gamma_ref, beta_ref, table_ref, o_hbm, *scr):
    n = len(_TILES)
    xbufs = scr[:n]
    obufs = xbufs
    insems, outsems = scr[n], scr[n + 1]
    d = dom_ref[0]
    gamma = gamma_ref[...]
    bias = beta_ref[...] + table_ref[pl.ds(d, 1), :]

    def in_copy(t):
        return pltpu.make_async_copy(
            x_hbm.at[pl.ds(_OFFS[t], _TILES[t])], xbufs[t], insems.at[t])

    def out_copy(t):
        return pltpu.make_async_copy(
            obufs[t], o_hbm.at[pl.ds(_OFFS[t], _TILES[t])], outsems.at[t])

    for t in range(n):
        in_copy(t).start()
    for t in range(n):
        in_copy(t).wait()
        obufs[t][...] = _ln_block(xbufs[t][...], gamma, bias)
        out_copy(t).start()
    for t in range(n):
        out_copy(t).wait()


def kernel(inputs, gamma, beta, inner_bias, domain):
    dom = jnp.asarray(domain, dtype=jnp.int32).reshape((1,))
    gamma2 = gamma.reshape(1, _DIM)
    beta2 = beta.reshape(1, _DIM)
    return pl.pallas_call(
        _mb_kernel,
        in_specs=[
            pl.BlockSpec(memory_space=pltpu.SMEM),
            pl.BlockSpec(memory_space=pl.ANY),
            pl.BlockSpec((1, _DIM), lambda: (0, 0)),
            pl.BlockSpec((1, _DIM), lambda: (0, 0)),
            pl.BlockSpec((_TABLE_ROWS, _DIM), lambda: (0, 0)),
        ],
        out_specs=pl.BlockSpec(memory_space=pl.ANY),
        out_shape=jax.ShapeDtypeStruct((_BATCH, _DIM), jnp.float32),
        scratch_shapes=(
            [pltpu.VMEM((t, _DIM), jnp.float32) for t in _TILES]
            + [pltpu.SemaphoreType.DMA((len(_TILES),)),
               pltpu.SemaphoreType.DMA((len(_TILES),))]
        ),
    )(dom, inputs, gamma2, beta2, inner_bias)
